# baseline (speedup 1.0000x reference)
import functools

import jax
import jax.numpy as jnp
from jax import lax
from jax.experimental import pallas as pl
from jax.experimental.pallas import tpu as pltpu
from jax.experimental.pallas import tpu_sc as plsc

B, S, D = 4, 8192, 1024
NW = 32
SPW = S // NW
CS = 8
NCH = SPW // CS
HALF = NCH // 2
NV = D // 16


def _sc_body(x_hbm, m_hbm, e_hbm, o_hbm,
             xb0, xb1, eb0, eb1, mbuf,
             in0, in1, es0, es1, ou0, ou1):
    cid = lax.axis_index("c")
    sid = lax.axis_index("s")
    wid = sid * 2 + cid
    s0 = wid * SPW

    for b in range(B):
        pltpu.sync_copy(m_hbm.at[b, pl.ds(s0, SPW)], mbuf.at[b])

    def start_in(c, xbuf, ebuf, isem, esem):
        s = s0 + c * CS
        pltpu.async_copy(e_hbm.at[pl.ds(s, CS), :], ebuf, esem)
        pltpu.async_copy(x_hbm.at[:, pl.ds(s, CS), :], xbuf, isem)

    def wait_in(xbuf, ebuf, isem, esem):
        pltpu.make_async_copy(e_hbm.at[pl.ds(0, CS), :], ebuf, esem).wait()
        pltpu.make_async_copy(x_hbm.at[:, pl.ds(0, CS), :], xbuf, isem).wait()

    def start_out(c, xbuf, osem):
        s = s0 + c * CS
        pltpu.async_copy(xbuf, o_hbm.at[:, pl.ds(s, CS), :], osem)

    def wait_out(xbuf, osem):
        pltpu.make_async_copy(xbuf, o_hbm.at[:, pl.ds(0, CS), :], osem).wait()

    def compute(mvecs, half, xbuf, ebuf):
        for r in range(CS):
            mfs = []
            for b in range(B):
                mv = mvecs[b][half * CS + r]
                mfs.append(jnp.where(mv == 0, 0.0, 1.0))

            @plsc.parallel_loop(0, NV, unroll=4)
            def _(j, r=r, mfs=mfs, xbuf=xbuf, ebuf=ebuf):
                sl = pl.ds(pl.multiple_of(j * 16, 16), 16)
                e = ebuf[r, sl]
                for b in range(B):
                    xbuf[b, r, sl] = (xbuf[b, r, sl] + e) * mfs[b]

    start_in(0, xb0, eb0, in0, es0)
    start_in(1, xb1, eb1, in1, es1)

    def body(t, _):
        c0 = 2 * t
        c1 = 2 * t + 1
        mvecs = [
            mbuf[b, pl.ds(pl.multiple_of(t * 2 * CS, 16), 16)] for b in range(B)
        ]
        wait_in(xb0, eb0, in0, es0)
        compute(mvecs, 0, xb0, eb0)
        start_out(c0, xb0, ou0)
        wait_in(xb1, eb1, in1, es1)
        compute(mvecs, 1, xb1, eb1)
        start_out(c1, xb1, ou1)

        @pl.when(t < HALF - 1)
        def _():
            wait_out(xb0, ou0)
            start_in(c0 + 2, xb0, eb0, in0, es0)
            wait_out(xb1, ou1)
            start_in(c1 + 2, xb1, eb1, in1, es1)

        return 0

    lax.fori_loop(0, HALF, body, 0)
    wait_out(xb0, ou0)
    wait_out(xb1, ou1)


def kernel(inputs, input_mask, pos_emb):
    run = functools.partial(
        pl.kernel,
        out_type=jax.ShapeDtypeStruct((B, S, D), jnp.float32),
        mesh=plsc.VectorSubcoreMesh(core_axis_name="c", subcore_axis_name="s"),
        scratch_types=[
            pltpu.VMEM((B, CS, D), jnp.float32),
            pltpu.VMEM((B, CS, D), jnp.float32),
            pltpu.VMEM((CS, D), jnp.float32),
            pltpu.VMEM((CS, D), jnp.float32),
            pltpu.VMEM((B, SPW), jnp.int32),
            pltpu.SemaphoreType.DMA,
            pltpu.SemaphoreType.DMA,
            pltpu.SemaphoreType.DMA,
            pltpu.SemaphoreType.DMA,
            pltpu.SemaphoreType.DMA,
            pltpu.SemaphoreType.DMA,
        ],
    )(_sc_body)
    return run(inputs, input_mask, pos_emb[:S])


# speedup vs baseline: 1.3340x; 1.3340x over previous
import jax
import jax.numpy as jnp
from jax import lax
from jax.experimental import pallas as pl
from jax.experimental.pallas import tpu as pltpu

B, S, D = 4, 8192, 1024
R = 512
NROWS = B * S
NCHUNK = NROWS // R
NEC = S // R
K = 4


def _body(x_hbm, m_hbm, e_hbm, o_hbm,
          ebuf, xbuf, obuf, mbuf,
          sem_e, sem_in, sem_m, sem_out):
    for c in range(NEC):
        pltpu.make_async_copy(
            e_hbm.at[pl.ds(c * R, R)], ebuf.at[pl.ds(c * R, R)], sem_e.at[c]
        ).start()

    def start_in(i, slot):
        pltpu.make_async_copy(
            x_hbm.at[pl.ds(i * R, R)], xbuf.at[slot], sem_in.at[slot]
        ).start()
        pltpu.make_async_copy(
            m_hbm.at[pl.ds(i * R, R)], mbuf.at[slot], sem_m.at[slot]
        ).start()

    for k in range(K):
        start_in(k, k)

    def step(i, _):
        slot = lax.rem(i, K)
        pltpu.make_async_copy(
            x_hbm.at[pl.ds(0, R)], xbuf.at[slot], sem_in.at[slot]
        ).wait()
        pltpu.make_async_copy(
            m_hbm.at[pl.ds(0, R)], mbuf.at[slot], sem_m.at[slot]
        ).wait()

        @pl.when(i < NEC)
        def _():
            pltpu.make_async_copy(
                e_hbm.at[pl.ds(0, R)], ebuf.at[pl.ds(0, R)], sem_e.at[i]
            ).wait()

        @pl.when(i >= K)
        def _():
            pltpu.make_async_copy(
                obuf.at[slot], o_hbm.at[pl.ds(0, R)], sem_out.at[slot]
            ).wait()

        ec = lax.rem(i, NEC)
        e = ebuf[pl.ds(ec * R, R), :]
        obuf[slot] = jnp.where(mbuf[slot] == 0, 0.0, xbuf[slot] + e)

        pltpu.make_async_copy(
            obuf.at[slot], o_hbm.at[pl.ds(i * R, R)], sem_out.at[slot]
        ).start(priority=1)

        @pl.when(i + K < NCHUNK)
        def _():
            start_in(i + K, slot)

        return 0

    lax.fori_loop(0, NCHUNK, step, 0)

    for j in range(K):
        slot = (NCHUNK - K + j) % K
        pltpu.make_async_copy(
            obuf.at[slot], o_hbm.at[pl.ds(0, R)], sem_out.at[slot]
        ).wait()


def kernel(inputs, input_mask, pos_emb):
    x = inputs.reshape(NROWS, D)
    m = input_mask.reshape(NROWS, 1)
    out = pl.pallas_call(
        _body,
        in_specs=[
            pl.BlockSpec(memory_space=pl.ANY),
            pl.BlockSpec(memory_space=pl.ANY),
            pl.BlockSpec(memory_space=pl.ANY),
        ],
        out_specs=pl.BlockSpec(memory_space=pl.ANY),
        out_shape=jax.ShapeDtypeStruct((NROWS, D), jnp.float32),
        scratch_shapes=[
            pltpu.VMEM((S, D), jnp.float32),
            pltpu.VMEM((K, R, D), jnp.float32),
            pltpu.VMEM((K, R, D), jnp.float32),
            pltpu.VMEM((K, R, 1), jnp.int32),
            pltpu.SemaphoreType.DMA((NEC,)),
            pltpu.SemaphoreType.DMA((K,)),
            pltpu.SemaphoreType.DMA((K,)),
            pltpu.SemaphoreType.DMA((K,)),
        ],
    )(x, m, pos_emb[:S])
    return out.reshape(B, S, D)
